# baseline (speedup 1.0000x reference)
import jax
import jax.numpy as jnp
from jax import lax
from jax.experimental import pallas as pl
from jax.experimental.pallas import tpu as pltpu
from jax.experimental.pallas import tpu_sc as plsc

_NC = 2
_NS = 16
_NW = _NC * _NS
_L = 16
_C = 80

_F32 = jnp.float32

_SC_PARAMS = pltpu.CompilerParams(needs_layout_passes=False)


def _sc_mesh():
    return plsc.VectorSubcoreMesh(
        core_axis_name="c", subcore_axis_name="s",
        num_cores=_NC, num_subcores=_NS)


def _wid():
    return lax.axis_index("c") * _NS + lax.axis_index("s")


def _deg_call(row1, chunks, npad):
    c = _C
    cpt = chunks // _NW
    rpt = npad // _NS
    kb = 25

    def body(row1_ref, z1_ref, ones_ref, degp_ref, onesv, idxr, degsh, sem,
             isem):
        cid = lax.axis_index("c")
        sid = lax.axis_index("s")
        w = _wid()
        base = w * cpt
        pltpu.sync_copy(z1_ref, degsh.at[pl.ds(sid * rpt, rpt)])
        pltpu.sync_copy(ones_ref, onesv)

        def iload(i, c2):
            pltpu.async_copy(row1_ref.at[pl.ds((base + i) * c, c)],
                             idxr.at[i], isem)
            return c2

        lax.fori_loop(0, cpt, iload, 0)

        def idrain(i, c2):
            pltpu.make_async_copy(row1_ref.at[pl.ds(base * c, c)],
                                  idxr.at[0], isem).wait()
            return c2

        lax.fori_loop(0, cpt, idrain, 0)
        plsc.subcore_barrier()

        def block(b, carry):
            def fire(i, c2):
                pltpu.async_copy(onesv, degsh.at[idxr.at[b * kb + i]], sem,
                                 add=True)
                return c2

            lax.fori_loop(0, kb, fire, 0)

            def drain(i, c2):
                pltpu.make_async_copy(onesv, degsh.at[idxr.at[0]], sem).wait()
                return c2

            lax.fori_loop(0, kb, drain, 0)
            return carry

        lax.fori_loop(0, cpt // kb, block, 0)
        plsc.subcore_barrier()
        pltpu.sync_copy(degsh.at[pl.ds(sid * rpt, rpt)],
                        degp_ref.at[cid, pl.ds(sid * rpt, rpt)])

    f = pl.kernel(
        body,
        out_type=jax.ShapeDtypeStruct((_NC, npad), _F32),
        mesh=_sc_mesh(),
        compiler_params=_SC_PARAMS,
        scratch_types=[
            pltpu.VMEM((c,), _F32),
            pltpu.VMEM((cpt, c), jnp.int32),
            pltpu.VMEM_SHARED((npad,), _F32),
            pltpu.SemaphoreType.DMA,
            pltpu.SemaphoreType.DMA,
        ],
    )
    return f(row1, jnp.zeros((rpt,), _F32), jnp.ones((c,), _F32))


def _drow_call(row1, chunks, dinvp):
    c = _C
    cpt = chunks // _NW
    e = chunks * c

    def body(row1_ref, dinv_ref, drow_ref, dinvv, idxr, dro, isem):
        w = _wid()
        base = w * cpt
        pltpu.sync_copy(dinv_ref, dinvv)

        def iload(i, c2):
            pltpu.async_copy(row1_ref.at[pl.ds((base + i) * c, c)],
                             idxr.at[i], isem)
            return c2

        lax.fori_loop(0, cpt, iload, 0)

        def idrain(i, c2):
            pltpu.make_async_copy(row1_ref.at[pl.ds(base * c, c)],
                                  idxr.at[0], isem).wait()
            return c2

        lax.fori_loop(0, cpt, idrain, 0)

        def step(i, carry):
            for r in range(c // _L):
                dro[pl.ds(i * c + r * _L, _L)] = plsc.load_gather(
                    dinvv, [idxr[i, pl.ds(r * _L, _L)]])
            return carry

        lax.fori_loop(0, cpt, step, 0, unroll=2)
        pltpu.sync_copy(dro, drow_ref.at[pl.ds(base * c, cpt * c)])

    f = pl.kernel(
        body,
        out_type=jax.ShapeDtypeStruct((e,), _F32),
        mesh=_sc_mesh(),
        compiler_params=_SC_PARAMS,
        scratch_types=[
            pltpu.VMEM((dinvp.shape[0],), _F32),
            pltpu.VMEM((cpt, c), jnp.int32),
            pltpu.VMEM((cpt * c,), _F32),
            pltpu.SemaphoreType.DMA,
        ],
    )
    return f(row1, dinvp)


def _agg_call(xs, els, row1, col1, chunks, npad):
    n, h = xs.shape
    c = _C
    cpt = chunks // _NW
    rpt = npad // _NS

    def body(xs_ref, el_ref, row1_ref, col1_ref, z2_ref, pout_ref,
             idxr, idxc, gv, accsh, gsem, irsem, icsem, esem, ssem):
        cid = lax.axis_index("c")
        sid = lax.axis_index("s")
        w = _wid()
        base = w * cpt
        pltpu.sync_copy(z2_ref, accsh.at[pl.ds(sid * rpt, rpt)])

        def load(i, k):
            pltpu.async_copy(row1_ref.at[pl.ds((base + i) * c, c)],
                             idxr.at[k], irsem.at[k])
            pltpu.async_copy(col1_ref.at[pl.ds((base + i) * c, c)],
                             idxc.at[k], icsem.at[k])
            pltpu.async_copy(el_ref.at[pl.ds((base + i) * c, c)], gv.at[k],
                             esem.at[k])

        def ga(i, k):
            pltpu.make_async_copy(row1_ref.at[pl.ds((base + i) * c, c)],
                                  idxr.at[k], irsem.at[k]).wait()
            pltpu.make_async_copy(el_ref.at[pl.ds((base + i) * c, c)],
                                  gv.at[k], esem.at[k]).wait()
            pltpu.async_copy(xs_ref.at[idxr.at[k]], gv.at[k], gsem.at[k],
                             add=True)

        load(0, 0)
        load(1, 1)
        ga(0, 0)
        plsc.subcore_barrier()

        def step(j, carry):
            for k in range(3):
                i = 3 * j + k
                k1 = (k + 1) % 3
                k2 = (k + 2) % 3

                @pl.when(i < cpt)
                def _():
                    pltpu.make_async_copy(xs_ref.at[idxr.at[k]], gv.at[k],
                                          gsem.at[k]).wait()

                    def rowstep(r, rc):
                        for q in range(h // _L):
                            s = pl.ds(q * _L, _L)
                            gv[k, r, s] = jnp.maximum(gv[k, r, s], 0.0)
                        return rc

                    lax.fori_loop(0, c, rowstep, 0, unroll=2)
                    pltpu.make_async_copy(col1_ref.at[pl.ds((base + i) * c, c)],
                                          idxc.at[k], icsem.at[k]).wait()
                    pltpu.async_copy(gv.at[k], accsh.at[idxc.at[k]],
                                     ssem.at[k], add=True)

                    @pl.when(i + 1 < cpt)
                    def _():
                        ga(i + 1, k1)

                    @pl.when((i >= 1) & (i + 2 < cpt))
                    def _():
                        pltpu.make_async_copy(gv.at[k2],
                                              accsh.at[idxc.at[k2]],
                                              ssem.at[k2]).wait()

                    @pl.when(i + 2 < cpt)
                    def _():
                        load(i + 2, k2)
            return carry

        lax.fori_loop(0, (cpt + 2) // 3, step, 0)
        for k in ((cpt - 3) % 3, (cpt - 2) % 3, (cpt - 1) % 3):
            pltpu.make_async_copy(gv.at[k], accsh.at[idxc.at[k]],
                                  ssem.at[k]).wait()
        plsc.subcore_barrier()
        for k in range(rpt // 128):
            r0 = sid * rpt + k * 128
            pltpu.sync_copy(accsh.at[pl.ds(r0, 128)],
                            pout_ref.at[cid, pl.ds(r0, 128)])

    f = pl.kernel(
        body,
        out_type=jax.ShapeDtypeStruct((_NC, npad, h), _F32),
        mesh=_sc_mesh(),
        compiler_params=_SC_PARAMS,
        scratch_types=[
            pltpu.VMEM((3, c), jnp.int32),
            pltpu.VMEM((3, c), jnp.int32),
            pltpu.VMEM((3, c, h), _F32),
            pltpu.VMEM_SHARED((npad, h), _F32),
            pltpu.SemaphoreType.DMA((3,)),
            pltpu.SemaphoreType.DMA((3,)),
            pltpu.SemaphoreType.DMA((3,)),
            pltpu.SemaphoreType.DMA((3,)),
            pltpu.SemaphoreType.DMA((3,)),
        ],
    )
    return f(xs, els, row1, col1, jnp.zeros((rpt, h), _F32))


def _pair_call(q, el3, row1, col1, chunks):
    n, h = q.shape
    c = _C
    cpt = chunks // _NW
    e = chunks * c

    def body(q_ref, el3_ref, row1_ref, col1_ref, out_ref, sout_ref,
             idxr, idxc, ev, sacc, g1sem, g2sem, irsem, icsem, esem, ssem):
        w = _wid()
        base = w * cpt

        def zacc(i, c2):
            for qq in range(h // _L):
                sacc[i, pl.ds(qq * _L, _L)] = jnp.zeros((_L,), _F32)
            return c2

        lax.fori_loop(0, 2, zacc, 0)

        def load(i, k):
            pltpu.async_copy(row1_ref.at[pl.ds((base + i) * c, c)],
                             idxr.at[k], irsem.at[k])
            pltpu.async_copy(col1_ref.at[pl.ds((base + i) * c, c)],
                             idxc.at[k], icsem.at[k])
            pltpu.async_copy(el3_ref.at[pl.ds((base + i) * c, c)], ev.at[k],
                             esem.at[k])

        def ga(i, k):
            pltpu.make_async_copy(row1_ref.at[pl.ds((base + i) * c, c)],
                                  idxr.at[k], irsem.at[k]).wait()
            pltpu.make_async_copy(col1_ref.at[pl.ds((base + i) * c, c)],
                                  idxc.at[k], icsem.at[k]).wait()
            pltpu.make_async_copy(el3_ref.at[pl.ds((base + i) * c, c)],
                                  ev.at[k], esem.at[k]).wait()
            pltpu.async_copy(q_ref.at[idxr.at[k]], ev.at[k], g1sem.at[k],
                             add=True)
            pltpu.async_copy(q_ref.at[idxc.at[k]], ev.at[k], g2sem.at[k],
                             add=True)

        load(0, 0)
        load(1, 1)
        ga(0, 0)

        def step(j, carry):
            for k in range(3):
                i = 3 * j + k
                k1 = (k + 1) % 3
                k2 = (k + 2) % 3

                @pl.when(i < cpt)
                def _():
                    pltpu.make_async_copy(q_ref.at[idxr.at[k]], ev.at[k],
                                          g1sem.at[k]).wait()
                    pltpu.make_async_copy(q_ref.at[idxc.at[k]], ev.at[k],
                                          g2sem.at[k]).wait()
                    pltpu.async_copy(ev.at[k],
                                     out_ref.at[pl.ds((base + i) * c, c)],
                                     ssem.at[k])

                    @pl.when(i + 1 < cpt)
                    def _():
                        ga(i + 1, k1)

                    for qq in range(h // _L):
                        s = pl.ds(qq * _L, _L)

                        def rowstep(r, acc):
                            a1, a2 = acc
                            y = jnp.maximum(ev[k, r, s], 0.0)
                            return (a1 + y, a2 + y * y)

                        a1, a2 = lax.fori_loop(
                            0, c, rowstep,
                            (jnp.zeros((_L,), _F32), jnp.zeros((_L,), _F32)),
                            unroll=4)
                        sacc[0, s] += a1
                        sacc[1, s] += a2

                    @pl.when((i >= 1) & (i + 2 < cpt))
                    def _():
                        pltpu.make_async_copy(
                            ev.at[k2], out_ref.at[pl.ds(base * c, c)],
                            ssem.at[k2]).wait()

                    @pl.when(i + 2 < cpt)
                    def _():
                        load(i + 2, k2)
            return carry

        lax.fori_loop(0, (cpt + 2) // 3, step, 0)
        for k in ((cpt - 3) % 3, (cpt - 2) % 3, (cpt - 1) % 3):
            pltpu.make_async_copy(ev.at[k], out_ref.at[pl.ds(base * c, c)],
                                  ssem.at[k]).wait()
        pltpu.sync_copy(sacc, sout_ref.at[w])

    f = pl.kernel(
        body,
        out_type=(jax.ShapeDtypeStruct((e, h), _F32),
                  jax.ShapeDtypeStruct((_NW, 2, h), _F32)),
        mesh=_sc_mesh(),
        compiler_params=_SC_PARAMS,
        scratch_types=[
            pltpu.VMEM((3, c), jnp.int32),
            pltpu.VMEM((3, c), jnp.int32),
            pltpu.VMEM((3, c, h), _F32),
            pltpu.VMEM((2, h), _F32),
            pltpu.SemaphoreType.DMA((3,)),
            pltpu.SemaphoreType.DMA((3,)),
            pltpu.SemaphoreType.DMA((3,)),
            pltpu.SemaphoreType.DMA((3,)),
            pltpu.SemaphoreType.DMA((3,)),
            pltpu.SemaphoreType.DMA((3,)),
        ],
    )
    return f(q, el3, row1, col1)


_BM = 512
_BME = 4096


def _emlp_call(ea, drow, We, be):
    e, de = ea.shape
    h = We.shape[1]
    grid = pl.cdiv(e, _BME)
    scaled = drow is not None

    def body(ea_ref, *rest):
        if scaled:
            dr_ref, w_ref, b_ref, o_ref = rest
        else:
            w_ref, b_ref, o_ref = rest
        a = ea_ref[...]
        raw = jnp.dot(a, w_ref[...], preferred_element_type=_F32) + b_ref[...]
        if scaled:
            raw = raw * dr_ref[...].reshape(_BME, 1)
        o_ref[...] = raw

    in_specs = [pl.BlockSpec((_BME, de), lambda i: (i, 0))]
    args = [ea]
    if scaled:
        in_specs.append(pl.BlockSpec((_BME,), lambda i: (i,)))
        args.append(drow)
    in_specs += [pl.BlockSpec((de, h), lambda i: (0, 0)),
                 pl.BlockSpec((1, h), lambda i: (0, 0))]
    args += [We, be.reshape(1, h)]
    return pl.pallas_call(
        body,
        grid=(grid,),
        in_specs=in_specs,
        out_specs=pl.BlockSpec((_BME, h), lambda i: (i, 0)),
        out_shape=jax.ShapeDtypeStruct((e, h), _F32),
    )(*args)


def _mm_call(z, W, b, scale, shift, prerelu=False, rowscale=None):
    m, k = z.shape
    h = W.shape[1]
    bm = _BME if m % _BME == 0 else _BM
    grid = pl.cdiv(m, bm)
    has_rs = rowscale is not None

    def body(z_ref, w_ref, b_ref, sc_ref, sh_ref, *rest):
        out_ref = rest[-1]
        zz = z_ref[...]
        if prerelu:
            zz = jnp.maximum(zz, 0.0)
        zz = zz * sc_ref[...] + sh_ref[...]
        acc = jnp.dot(zz, w_ref[...], preferred_element_type=_F32) + b_ref[...]
        if has_rs:
            acc = acc * rest[0][...]
        out_ref[...] = acc

    in_specs = [pl.BlockSpec((bm, k), lambda i: (i, 0)),
                pl.BlockSpec((k, h), lambda i: (0, 0)),
                pl.BlockSpec((1, h), lambda i: (0, 0)),
                pl.BlockSpec((1, h), lambda i: (0, 0)),
                pl.BlockSpec((1, h), lambda i: (0, 0))]
    args = [z, W, b.reshape(1, h), scale.reshape(1, h), shift.reshape(1, h)]
    if has_rs:
        in_specs.append(pl.BlockSpec((bm, 1), lambda i: (i, 0)))
        args.append(rowscale)
    return pl.pallas_call(
        body,
        grid=(grid,),
        in_specs=in_specs,
        out_specs=pl.BlockSpec((bm, h), lambda i: (i, 0)),
        out_shape=jax.ShapeDtypeStruct((m, h), _F32),
    )(*args)


def _stats_call(p, rowscale=None, with_raw=True):
    np_, m, h = p.shape
    bm = _BME if m % _BME == 0 else _BM
    grid = pl.cdiv(m, bm)
    has_rs = rowscale is not None

    def body(p_ref, *rest):
        if has_rs:
            rs_ref, rest = rest[0], rest[1:]
        if with_raw:
            raw_ref, s1_ref, s2_ref = rest
        else:
            s1_ref, s2_ref = rest
        i = pl.program_id(0)
        r = p_ref[0]
        for j in range(1, np_):
            r = r + p_ref[j]
        if has_rs:
            r = r * rs_ref[...]
        if with_raw:
            raw_ref[...] = r
        y = jnp.maximum(r, 0.0)
        rows = lax.broadcasted_iota(jnp.int32, (bm, h), 0) + i * bm
        ym = jnp.where(rows < m, y, 0.0)

        @pl.when(i == 0)
        def _init():
            s1_ref[...] = jnp.zeros_like(s1_ref)
            s2_ref[...] = jnp.zeros_like(s2_ref)

        s1_ref[...] += jnp.sum(ym, axis=0, keepdims=True)
        s2_ref[...] += jnp.sum(ym * ym, axis=0, keepdims=True)

    in_specs = [pl.BlockSpec((np_, bm, h), lambda i: (0, i, 0))]
    args = [p]
    if has_rs:
        in_specs.append(pl.BlockSpec((bm, 1), lambda i: (i, 0)))
        args.append(rowscale)
    sspec = pl.BlockSpec((1, h), lambda i: (0, 0))
    sshape = jax.ShapeDtypeStruct((1, h), _F32)
    out_specs = [sspec, sspec]
    out_shape = [sshape, sshape]
    if with_raw:
        out_specs = [pl.BlockSpec((bm, h), lambda i: (i, 0))] + out_specs
        out_shape = [jax.ShapeDtypeStruct((m, h), _F32)] + out_shape
    return pl.pallas_call(
        body,
        grid=(grid,),
        in_specs=in_specs,
        out_specs=out_specs,
        out_shape=out_shape,
    )(*args)


def _affine(s1, s2, m, g, bt):
    mean = s1[0] / m
    var = s2[0] / m - mean * mean
    scale = g * lax.rsqrt(var + 1e-5)
    shift = bt - mean * scale
    return scale, shift


def kernel(x, edge_index, edge_attr, W1, b1, We1, be1, g1, bt1,
           W2, b2, We2, be2, g2, bt2,
           W3, b3, We3, be3, Wu3, bu3, g3, bt3, ge3, bte3,
           Wnode, bnode, Wedge, bedge):
    n, din = x.shape
    e = edge_index.shape[1]
    h = W1.shape[1]
    npad = ((n + 64 * _NS - 1) // (64 * _NS)) * (64 * _NS)
    chunks = e // _C
    cpt = chunks // _NW

    row1 = edge_index[0]
    col1 = edge_index[1]

    degp = _deg_call(row1, chunks, npad)
    dinv = lax.rsqrt(degp[0, :n] + degp[1, :n] + 1.0)
    dinvp = jnp.concatenate([dinv, jnp.ones((npad - n,), _F32)])
    dinv2 = dinv.reshape(n, 1)
    drow = _drow_call(row1, chunks, dinvp)

    e1s = _emlp_call(edge_attr, drow, We1, be1)
    e2s = _emlp_call(edge_attr, drow, We2, be2)
    e3s = _emlp_call(edge_attr, drow, We3, be3)
    e3r = _emlp_call(edge_attr, None, We3, be3)

    ones = jnp.ones((h,), _F32)
    zeros = jnp.zeros((h,), _F32)

    xs1 = _mm_call(x, W1, b1, ones, zeros, rowscale=dinv2)
    p1 = _agg_call(xs1, e1s, row1, col1, chunks, npad)
    raw1, s11, s21 = _stats_call(p1[:, :n], rowscale=dinv2)
    sc1, sh1 = _affine(s11, s21, n, g1, bt1)

    xs2 = _mm_call(raw1, W2, b2, sc1, sh1, prerelu=True, rowscale=dinv2)
    p2 = _agg_call(xs2, e2s, row1, col1, chunks, npad)
    raw2, s12, s22 = _stats_call(p2[:, :n], rowscale=dinv2)
    sc2, sh2 = _affine(s12, s22, n, g2, bt2)

    xs3 = _mm_call(raw2, W3, b3, sc2, sh2, prerelu=True, rowscale=dinv2)
    p3 = _agg_call(xs3, e3s, row1, col1, chunks, npad)
    raw3, s13, s23 = _stats_call(p3[:, :n], rowscale=dinv2)
    sc3, sh3 = _affine(s13, s23, n, g3, bt3)

    q = _mm_call(raw3, Wu3, 0.5 * bu3, ones, zeros)
    enew, souts = _pair_call(q, e3r, row1, col1, chunks)
    stot = jnp.sum(souts, axis=0)
    sce, she = _affine(stot[0:1], stot[1:2], e, ge3, bte3)

    out_h = _mm_call(raw3, Wnode, bnode, sc3, sh3, prerelu=True)
    out_e = _mm_call(enew, Wedge, bedge, sce, she, prerelu=True)
    return (out_h, out_e)


# speedup vs baseline: 1.0352x; 1.0352x over previous
import jax
import jax.numpy as jnp
from jax import lax
from jax.experimental import pallas as pl
from jax.experimental.pallas import tpu as pltpu
from jax.experimental.pallas import tpu_sc as plsc

_NC = 2
_NS = 16
_NW = _NC * _NS
_L = 16
_C = 80

_F32 = jnp.float32

_SC_PARAMS = pltpu.CompilerParams(needs_layout_passes=False)


def _sc_mesh():
    return plsc.VectorSubcoreMesh(
        core_axis_name="c", subcore_axis_name="s",
        num_cores=_NC, num_subcores=_NS)


def _wid():
    return lax.axis_index("c") * _NS + lax.axis_index("s")


def _deg_call(row1, chunks, npad):
    c = _C
    cpt = chunks // _NW
    rpt = npad // _NS
    kb = 25

    def body(row1_ref, z1_ref, ones_ref, degp_ref, onesv, idxr, degsh, sem,
             isem):
        cid = lax.axis_index("c")
        sid = lax.axis_index("s")
        w = _wid()
        base = w * cpt
        pltpu.sync_copy(z1_ref, degsh.at[pl.ds(sid * rpt, rpt)])
        pltpu.sync_copy(ones_ref, onesv)

        def iload(i, c2):
            pltpu.async_copy(row1_ref.at[pl.ds((base + i) * c, c)],
                             idxr.at[i], isem)
            return c2

        lax.fori_loop(0, cpt, iload, 0)

        def idrain(i, c2):
            pltpu.make_async_copy(row1_ref.at[pl.ds(base * c, c)],
                                  idxr.at[0], isem).wait()
            return c2

        lax.fori_loop(0, cpt, idrain, 0)
        plsc.subcore_barrier()

        def block(b, carry):
            def fire(i, c2):
                pltpu.async_copy(onesv, degsh.at[idxr.at[b * kb + i]], sem,
                                 add=True)
                return c2

            lax.fori_loop(0, kb, fire, 0)

            def drain(i, c2):
                pltpu.make_async_copy(onesv, degsh.at[idxr.at[0]], sem).wait()
                return c2

            lax.fori_loop(0, kb, drain, 0)
            return carry

        lax.fori_loop(0, cpt // kb, block, 0)
        plsc.subcore_barrier()
        pltpu.sync_copy(degsh.at[pl.ds(sid * rpt, rpt)],
                        degp_ref.at[cid, pl.ds(sid * rpt, rpt)])

    f = pl.kernel(
        body,
        out_type=jax.ShapeDtypeStruct((_NC, npad), _F32),
        mesh=_sc_mesh(),
        compiler_params=_SC_PARAMS,
        scratch_types=[
            pltpu.VMEM((c,), _F32),
            pltpu.VMEM((cpt, c), jnp.int32),
            pltpu.VMEM_SHARED((npad,), _F32),
            pltpu.SemaphoreType.DMA,
            pltpu.SemaphoreType.DMA,
        ],
    )
    return f(row1, jnp.zeros((rpt,), _F32), jnp.ones((c,), _F32))


def _drow_call(row1, chunks, dinvp):
    c = _C
    cpt = chunks // _NW
    e = chunks * c

    def body(row1_ref, dinv_ref, drow_ref, dinvv, idxr, dro, isem):
        w = _wid()
        base = w * cpt
        pltpu.sync_copy(dinv_ref, dinvv)

        def iload(i, c2):
            pltpu.async_copy(row1_ref.at[pl.ds((base + i) * c, c)],
                             idxr.at[i], isem)
            return c2

        lax.fori_loop(0, cpt, iload, 0)

        def idrain(i, c2):
            pltpu.make_async_copy(row1_ref.at[pl.ds(base * c, c)],
                                  idxr.at[0], isem).wait()
            return c2

        lax.fori_loop(0, cpt, idrain, 0)

        def step(i, carry):
            for r in range(c // _L):
                dro[pl.ds(i * c + r * _L, _L)] = plsc.load_gather(
                    dinvv, [idxr[i, pl.ds(r * _L, _L)]])
            return carry

        lax.fori_loop(0, cpt, step, 0, unroll=2)
        pltpu.sync_copy(dro, drow_ref.at[pl.ds(base * c, cpt * c)])

    f = pl.kernel(
        body,
        out_type=jax.ShapeDtypeStruct((e,), _F32),
        mesh=_sc_mesh(),
        compiler_params=_SC_PARAMS,
        scratch_types=[
            pltpu.VMEM((dinvp.shape[0],), _F32),
            pltpu.VMEM((cpt, c), jnp.int32),
            pltpu.VMEM((cpt * c,), _F32),
            pltpu.SemaphoreType.DMA,
        ],
    )
    return f(row1, dinvp)


def _agg_call(xs, els, row1, col1, chunks, npad):
    n, h = xs.shape
    c = _C
    cpt = chunks // _NW
    rpt = npad // _NS

    def body(xs_ref, el_ref, row1_ref, col1_ref, z2_ref, pout_ref,
             idxr, idxc, gv, accsh, gsem, irsem, icsem, esem, ssem):
        cid = lax.axis_index("c")
        sid = lax.axis_index("s")
        w = _wid()
        base = w * cpt
        pltpu.sync_copy(z2_ref, accsh.at[pl.ds(sid * rpt, rpt)])

        def load(i, k):
            pltpu.async_copy(row1_ref.at[pl.ds((base + i) * c, c)],
                             idxr.at[k], irsem.at[k])
            pltpu.async_copy(col1_ref.at[pl.ds((base + i) * c, c)],
                             idxc.at[k], icsem.at[k])
            pltpu.async_copy(el_ref.at[pl.ds((base + i) * c, c)], gv.at[k],
                             esem.at[k])

        def ga(i, k):
            pltpu.make_async_copy(row1_ref.at[pl.ds((base + i) * c, c)],
                                  idxr.at[k], irsem.at[k]).wait()
            pltpu.make_async_copy(el_ref.at[pl.ds((base + i) * c, c)],
                                  gv.at[k], esem.at[k]).wait()
            pltpu.async_copy(xs_ref.at[idxr.at[k]], gv.at[k], gsem.at[k],
                             add=True)

        load(0, 0)
        load(1, 1)
        ga(0, 0)
        plsc.subcore_barrier()

        def step(j, carry):
            for k in range(3):
                i = 3 * j + k
                k1 = (k + 1) % 3
                k2 = (k + 2) % 3

                @pl.when(i < cpt)
                def _():
                    pltpu.make_async_copy(xs_ref.at[idxr.at[k]], gv.at[k],
                                          gsem.at[k]).wait()

                    def rowstep(r, rc):
                        for q in range(h // _L):
                            s = pl.ds(q * _L, _L)
                            gv[k, r, s] = jnp.maximum(gv[k, r, s], 0.0)
                        return rc

                    lax.fori_loop(0, c, rowstep, 0, unroll=2)
                    pltpu.make_async_copy(col1_ref.at[pl.ds((base + i) * c, c)],
                                          idxc.at[k], icsem.at[k]).wait()
                    pltpu.async_copy(gv.at[k], accsh.at[idxc.at[k]],
                                     ssem.at[k], add=True)

                    @pl.when(i + 1 < cpt)
                    def _():
                        ga(i + 1, k1)

                    @pl.when((i >= 1) & (i + 2 < cpt))
                    def _():
                        pltpu.make_async_copy(gv.at[k2],
                                              accsh.at[idxc.at[k2]],
                                              ssem.at[k2]).wait()

                    @pl.when(i + 2 < cpt)
                    def _():
                        load(i + 2, k2)
            return carry

        lax.fori_loop(0, (cpt + 2) // 3, step, 0)
        for k in ((cpt - 3) % 3, (cpt - 2) % 3, (cpt - 1) % 3):
            pltpu.make_async_copy(gv.at[k], accsh.at[idxc.at[k]],
                                  ssem.at[k]).wait()
        plsc.subcore_barrier()
        for k in range(rpt // 128):
            r0 = sid * rpt + k * 128
            pltpu.sync_copy(accsh.at[pl.ds(r0, 128)],
                            pout_ref.at[cid, pl.ds(r0, 128)])

    f = pl.kernel(
        body,
        out_type=jax.ShapeDtypeStruct((_NC, npad, h), _F32),
        mesh=_sc_mesh(),
        compiler_params=_SC_PARAMS,
        scratch_types=[
            pltpu.VMEM((3, c), jnp.int32),
            pltpu.VMEM((3, c), jnp.int32),
            pltpu.VMEM((3, c, h), _F32),
            pltpu.VMEM_SHARED((npad, h), _F32),
            pltpu.SemaphoreType.DMA((3,)),
            pltpu.SemaphoreType.DMA((3,)),
            pltpu.SemaphoreType.DMA((3,)),
            pltpu.SemaphoreType.DMA((3,)),
            pltpu.SemaphoreType.DMA((3,)),
        ],
    )
    return f(xs, els, row1, col1, jnp.zeros((rpt, h), _F32))


def _pair_call(q, el3, row1, col1, chunks):
    n, h = q.shape
    c = _C
    cpt = chunks // _NW
    e = chunks * c

    def body(q_ref, el3_ref, row1_ref, col1_ref, out_ref, sout_ref,
             idxr, idxc, ev, sacc, g1sem, g2sem, irsem, icsem, esem, ssem):
        w = _wid()
        base = w * cpt

        def zacc(i, c2):
            for qq in range(h // _L):
                sacc[i, pl.ds(qq * _L, _L)] = jnp.zeros((_L,), _F32)
            return c2

        lax.fori_loop(0, 2, zacc, 0)

        def load(i, k):
            pltpu.async_copy(row1_ref.at[pl.ds((base + i) * c, c)],
                             idxr.at[k], irsem.at[k])
            pltpu.async_copy(col1_ref.at[pl.ds((base + i) * c, c)],
                             idxc.at[k], icsem.at[k])
            pltpu.async_copy(el3_ref.at[pl.ds((base + i) * c, c)], ev.at[k],
                             esem.at[k])

        def ga(i, k):
            pltpu.make_async_copy(row1_ref.at[pl.ds((base + i) * c, c)],
                                  idxr.at[k], irsem.at[k]).wait()
            pltpu.make_async_copy(col1_ref.at[pl.ds((base + i) * c, c)],
                                  idxc.at[k], icsem.at[k]).wait()
            pltpu.make_async_copy(el3_ref.at[pl.ds((base + i) * c, c)],
                                  ev.at[k], esem.at[k]).wait()
            pltpu.async_copy(q_ref.at[idxr.at[k]], ev.at[k], g1sem.at[k],
                             add=True)
            pltpu.async_copy(q_ref.at[idxc.at[k]], ev.at[k], g2sem.at[k],
                             add=True)

        load(0, 0)
        load(1, 1)
        ga(0, 0)

        def step(j, carry):
            for k in range(3):
                i = 3 * j + k
                k1 = (k + 1) % 3
                k2 = (k + 2) % 3

                @pl.when(i < cpt)
                def _():
                    pltpu.make_async_copy(q_ref.at[idxr.at[k]], ev.at[k],
                                          g1sem.at[k]).wait()
                    pltpu.make_async_copy(q_ref.at[idxc.at[k]], ev.at[k],
                                          g2sem.at[k]).wait()
                    pltpu.async_copy(ev.at[k],
                                     out_ref.at[pl.ds((base + i) * c, c)],
                                     ssem.at[k])

                    @pl.when(i + 1 < cpt)
                    def _():
                        ga(i + 1, k1)

                    for qq in range(h // _L):
                        s = pl.ds(qq * _L, _L)

                        def rowstep(r, acc):
                            a1, a2 = acc
                            y = jnp.maximum(ev[k, r, s], 0.0)
                            return (a1 + y, a2 + y * y)

                        a1, a2 = lax.fori_loop(
                            0, c, rowstep,
                            (jnp.zeros((_L,), _F32), jnp.zeros((_L,), _F32)),
                            unroll=4)
                        sacc[0, s] += a1
                        sacc[1, s] += a2

                    @pl.when((i >= 1) & (i + 2 < cpt))
                    def _():
                        pltpu.make_async_copy(
                            ev.at[k2], out_ref.at[pl.ds(base * c, c)],
                            ssem.at[k2]).wait()

                    @pl.when(i + 2 < cpt)
                    def _():
                        load(i + 2, k2)
            return carry

        lax.fori_loop(0, (cpt + 2) // 3, step, 0)
        for k in ((cpt - 3) % 3, (cpt - 2) % 3, (cpt - 1) % 3):
            pltpu.make_async_copy(ev.at[k], out_ref.at[pl.ds(base * c, c)],
                                  ssem.at[k]).wait()
        pltpu.sync_copy(sacc, sout_ref.at[w])

    f = pl.kernel(
        body,
        out_type=(jax.ShapeDtypeStruct((e, h), _F32),
                  jax.ShapeDtypeStruct((_NW, 2, h), _F32)),
        mesh=_sc_mesh(),
        compiler_params=_SC_PARAMS,
        scratch_types=[
            pltpu.VMEM((3, c), jnp.int32),
            pltpu.VMEM((3, c), jnp.int32),
            pltpu.VMEM((3, c, h), _F32),
            pltpu.VMEM((2, h), _F32),
            pltpu.SemaphoreType.DMA((3,)),
            pltpu.SemaphoreType.DMA((3,)),
            pltpu.SemaphoreType.DMA((3,)),
            pltpu.SemaphoreType.DMA((3,)),
            pltpu.SemaphoreType.DMA((3,)),
            pltpu.SemaphoreType.DMA((3,)),
        ],
    )
    return f(q, el3, row1, col1)


_BM = 512
_BME = 2048


def _emlp_call(ea, drow, We, be):
    e, de = ea.shape
    h = We.shape[1]
    grid = pl.cdiv(e, _BME)
    scaled = drow is not None

    def body(ea_ref, *rest):
        if scaled:
            dr_ref, w_ref, b_ref, o_ref = rest
        else:
            w_ref, b_ref, o_ref = rest
        a = ea_ref[...]
        raw = jnp.dot(a, w_ref[...], preferred_element_type=_F32) + b_ref[...]
        if scaled:
            raw = raw * dr_ref[...].reshape(_BME, 1)
        o_ref[...] = raw

    in_specs = [pl.BlockSpec((_BME, de), lambda i: (i, 0))]
    args = [ea]
    if scaled:
        in_specs.append(pl.BlockSpec((_BME,), lambda i: (i,)))
        args.append(drow)
    in_specs += [pl.BlockSpec((de, h), lambda i: (0, 0)),
                 pl.BlockSpec((1, h), lambda i: (0, 0))]
    args += [We, be.reshape(1, h)]
    return pl.pallas_call(
        body,
        grid=(grid,),
        in_specs=in_specs,
        out_specs=pl.BlockSpec((_BME, h), lambda i: (i, 0)),
        out_shape=jax.ShapeDtypeStruct((e, h), _F32),
    )(*args)


def _mm_call(z, W, b, scale, shift, prerelu=False, rowscale=None):
    m, k = z.shape
    h = W.shape[1]
    bm = _BME if m % _BME == 0 else _BM
    grid = pl.cdiv(m, bm)
    has_rs = rowscale is not None

    def body(z_ref, w_ref, b_ref, sc_ref, sh_ref, *rest):
        out_ref = rest[-1]
        zz = z_ref[...]
        if prerelu:
            zz = jnp.maximum(zz, 0.0)
        zz = zz * sc_ref[...] + sh_ref[...]
        acc = jnp.dot(zz, w_ref[...], preferred_element_type=_F32) + b_ref[...]
        if has_rs:
            acc = acc * rest[0][...]
        out_ref[...] = acc

    in_specs = [pl.BlockSpec((bm, k), lambda i: (i, 0)),
                pl.BlockSpec((k, h), lambda i: (0, 0)),
                pl.BlockSpec((1, h), lambda i: (0, 0)),
                pl.BlockSpec((1, h), lambda i: (0, 0)),
                pl.BlockSpec((1, h), lambda i: (0, 0))]
    args = [z, W, b.reshape(1, h), scale.reshape(1, h), shift.reshape(1, h)]
    if has_rs:
        in_specs.append(pl.BlockSpec((bm, 1), lambda i: (i, 0)))
        args.append(rowscale)
    return pl.pallas_call(
        body,
        grid=(grid,),
        in_specs=in_specs,
        out_specs=pl.BlockSpec((bm, h), lambda i: (i, 0)),
        out_shape=jax.ShapeDtypeStruct((m, h), _F32),
    )(*args)


def _stats_call(p, rowscale=None, with_raw=True, nvalid=None):
    np_, m, h = p.shape
    nv = m if nvalid is None else nvalid
    bm = _BME if m % _BME == 0 else _BM
    grid = pl.cdiv(m, bm)
    has_rs = rowscale is not None

    def body(p_ref, *rest):
        if has_rs:
            rs_ref, rest = rest[0], rest[1:]
        if with_raw:
            raw_ref, s1_ref, s2_ref = rest
        else:
            s1_ref, s2_ref = rest
        i = pl.program_id(0)
        r = p_ref[0]
        for j in range(1, np_):
            r = r + p_ref[j]
        if has_rs:
            r = r * rs_ref[...]
        if with_raw:
            raw_ref[...] = r
        y = jnp.maximum(r, 0.0)
        rows = lax.broadcasted_iota(jnp.int32, (bm, h), 0) + i * bm
        ym = jnp.where(rows < nv, y, 0.0)

        @pl.when(i == 0)
        def _init():
            s1_ref[...] = jnp.zeros_like(s1_ref)
            s2_ref[...] = jnp.zeros_like(s2_ref)

        s1_ref[...] += jnp.sum(ym, axis=0, keepdims=True)
        s2_ref[...] += jnp.sum(ym * ym, axis=0, keepdims=True)

    in_specs = [pl.BlockSpec((np_, bm, h), lambda i: (0, i, 0))]
    args = [p]
    if has_rs:
        in_specs.append(pl.BlockSpec((bm, 1), lambda i: (i, 0)))
        args.append(rowscale)
    sspec = pl.BlockSpec((1, h), lambda i: (0, 0))
    sshape = jax.ShapeDtypeStruct((1, h), _F32)
    out_specs = [sspec, sspec]
    out_shape = [sshape, sshape]
    if with_raw:
        out_specs = [pl.BlockSpec((bm, h), lambda i: (i, 0))] + out_specs
        out_shape = [jax.ShapeDtypeStruct((m, h), _F32)] + out_shape
    return pl.pallas_call(
        body,
        grid=(grid,),
        in_specs=in_specs,
        out_specs=out_specs,
        out_shape=out_shape,
    )(*args)


def _affine(s1, s2, m, g, bt):
    mean = s1[0] / m
    var = s2[0] / m - mean * mean
    scale = g * lax.rsqrt(var + 1e-5)
    shift = bt - mean * scale
    return scale, shift


def kernel(x, edge_index, edge_attr, W1, b1, We1, be1, g1, bt1,
           W2, b2, We2, be2, g2, bt2,
           W3, b3, We3, be3, Wu3, bu3, g3, bt3, ge3, bte3,
           Wnode, bnode, Wedge, bedge):
    n, din = x.shape
    e = edge_index.shape[1]
    h = W1.shape[1]
    npad = ((n + 64 * _NS - 1) // (64 * _NS)) * (64 * _NS)
    chunks = e // _C
    cpt = chunks // _NW

    row1 = edge_index[0]
    col1 = edge_index[1]

    degp = _deg_call(row1, chunks, npad)
    dinv = lax.rsqrt(degp[0, :n] + degp[1, :n] + 1.0)
    dinvp = jnp.concatenate([dinv, jnp.ones((npad - n,), _F32)])
    dinv2 = dinv.reshape(n, 1)
    drow = _drow_call(row1, chunks, dinvp)

    e1s = _emlp_call(edge_attr, drow, We1, be1)
    e2s = _emlp_call(edge_attr, drow, We2, be2)
    e3s = _emlp_call(edge_attr, drow, We3, be3)
    e3r = _emlp_call(edge_attr, None, We3, be3)

    ones = jnp.ones((h,), _F32)
    zeros = jnp.zeros((h,), _F32)

    dinvp2 = dinvp.reshape(npad, 1)
    xs1 = _mm_call(x, W1, b1, ones, zeros, rowscale=dinv2)
    p1 = _agg_call(xs1, e1s, row1, col1, chunks, npad)
    raw1, s11, s21 = _stats_call(p1, rowscale=dinvp2, nvalid=n)
    sc1, sh1 = _affine(s11, s21, n, g1, bt1)

    xs2 = _mm_call(raw1, W2, b2, sc1, sh1, prerelu=True, rowscale=dinvp2)
    p2 = _agg_call(xs2, e2s, row1, col1, chunks, npad)
    raw2, s12, s22 = _stats_call(p2, rowscale=dinvp2, nvalid=n)
    sc2, sh2 = _affine(s12, s22, n, g2, bt2)

    xs3 = _mm_call(raw2, W3, b3, sc2, sh2, prerelu=True, rowscale=dinvp2)
    p3 = _agg_call(xs3, e3s, row1, col1, chunks, npad)
    raw3, s13, s23 = _stats_call(p3, rowscale=dinvp2, nvalid=n)
    sc3, sh3 = _affine(s13, s23, n, g3, bt3)

    q = _mm_call(raw3, Wu3, 0.5 * bu3, ones, zeros)
    enew, souts = _pair_call(q, e3r, row1, col1, chunks)
    stot = jnp.sum(souts, axis=0)
    sce, she = _affine(stot[0:1], stot[1:2], e, ge3, bte3)

    out_h = _mm_call(raw3, Wnode, bnode, sc3, sh3, prerelu=True)[:n]
    out_e = _mm_call(enew, Wedge, bedge, sce, she, prerelu=True)
    return (out_h, out_e)
